# baseline (speedup 1.0000x reference)
import functools

import jax
import jax.numpy as jnp
from jax import lax
from jax.experimental import pallas as pl
from jax.experimental.pallas import tpu as pltpu
from jax.experimental.pallas import tpu_sc as plsc

N_SIDE = 1000000
S_EMB = 32
D_SIDE = 64
AA_H = 152
B = 4096

_PACK = 128 // S_EMB
_ROWS = N_SIDE // _PACK
_REP = (AA_H * D_SIDE) // 128


@functools.cache
def _make_sc_gather():
    info = plsc.get_sparse_core_info()
    nw = info.num_cores * info.num_subcores
    b_per_w = B // nw
    mesh = plsc.VectorSubcoreMesh(core_axis_name="c", subcore_axis_name="s")

    @functools.partial(
        pl.kernel,
        mesh=mesh,
        out_type=jax.ShapeDtypeStruct((B, 128), jnp.float32),
        scratch_types=[
            pltpu.VMEM((b_per_w,), jnp.int32),
            pltpu.VMEM((b_per_w,), jnp.int32),
            pltpu.VMEM((b_per_w, 128), jnp.float32),
            pltpu.SemaphoreType.DMA,
        ],
    )
    def gather_k(idx_hbm, table_hbm, out_hbm, idx_v, idx_hi_v, rows_v, sem):
        wid = lax.axis_index("s") * info.num_cores + lax.axis_index("c")
        base = wid * b_per_w
        pltpu.sync_copy(idx_hbm.at[pl.ds(base, b_per_w)], idx_v)
        for j in range(b_per_w // 16):
            sl = pl.ds(j * 16, 16)
            idx_hi_v[sl] = idx_v[sl] >> 2
        pltpu.async_copy(table_hbm.at[idx_hi_v], rows_v, sem).wait()
        pltpu.sync_copy(rows_v, out_hbm.at[pl.ds(base, b_per_w)])

    return gather_k


def _mlp_tile_body(emb_ref, sel_ref, w1s_ref, b1_ref, gamma_ref, beta_ref,
                   w2t_ref, b2_ref, out_ref):
    emb = emb_ref[...]
    sel = sel_ref[...] & (_PACK - 1)
    lane_grp = lax.broadcasted_iota(jnp.int32, emb.shape, 1) >> 5
    mask = (lane_grp == sel).astype(jnp.float32)
    h = jnp.dot(emb * mask, w1s_ref[...], preferred_element_type=jnp.float32)
    h = h + b1_ref[...]
    mu = jnp.mean(h, axis=1, keepdims=True)
    var = jnp.mean((h - mu) ** 2, axis=1, keepdims=True)
    h = (h - mu) * lax.rsqrt(var + 1e-5) * gamma_ref[...] + beta_ref[...]
    h = jnp.maximum(h, 0.0)
    h = jnp.dot(h, w2t_ref[...], preferred_element_type=jnp.float32)
    h = h + b2_ref[...]
    h2 = jnp.concatenate([h, h], axis=1)
    for a in range(_REP):
        out_ref[:, a * 128:(a + 1) * 128] = h2


def _tc_mlp_tile(emb, idx2d, w1s, b1, gamma, beta, w2t, b2, bm=256):
    grid = B // bm
    return pl.pallas_call(
        _mlp_tile_body,
        grid=(grid,),
        in_specs=[
            pl.BlockSpec((bm, 128), lambda i: (i, 0)),
            pl.BlockSpec((bm, 1), lambda i: (i, 0)),
            pl.BlockSpec((128, D_SIDE), lambda i: (0, 0)),
            pl.BlockSpec((1, D_SIDE), lambda i: (0, 0)),
            pl.BlockSpec((1, D_SIDE), lambda i: (0, 0)),
            pl.BlockSpec((1, D_SIDE), lambda i: (0, 0)),
            pl.BlockSpec((D_SIDE, D_SIDE), lambda i: (0, 0)),
            pl.BlockSpec((1, D_SIDE), lambda i: (0, 0)),
        ],
        out_specs=pl.BlockSpec((bm, AA_H * D_SIDE), lambda i: (i, 0)),
        out_shape=jax.ShapeDtypeStruct((B, AA_H * D_SIDE), jnp.float32),
    )(emb, idx2d, w1s, b1, gamma, beta, w2t, b2)


def kernel(side, table, W1, b1, gamma, beta, W2, b2):
    idx = side.astype(jnp.int32)
    table128 = table.reshape(_ROWS, 128)
    emb = _make_sc_gather()(idx, table128)
    out2d = _tc_mlp_tile(
        emb,
        idx.reshape(B, 1),
        jnp.tile(W1.T, (_PACK, 1)),
        b1.reshape(1, D_SIDE),
        gamma.reshape(1, D_SIDE),
        beta.reshape(1, D_SIDE),
        W2.T,
        b2.reshape(1, D_SIDE),
    )
    return out2d.reshape(B, AA_H, D_SIDE)
